# baseline (speedup 1.0000x reference)
import functools

import jax
import jax.numpy as jnp
from jax import lax
from jax.experimental import pallas as pl
from jax.experimental.pallas import tpu as pltpu
from jax.experimental.pallas import tpu_sc as plsc

_H = 1024
_W = 1024
_F = 64
_HID = 128
_OUT = 64
_NB = 16
_N = 65536

_NC = 2
_NS = 16
_NW = _NC * _NS
_LANES = 16

_CH = 128
_GROUPS = _CH // _LANES
_PER_TILE = _NB * _N // _NW
_NCHUNK = _PER_TILE // _CH
_HALF = _PER_TILE

_TCB = 64


def _sc_sample_body(coords_hbm, table_hbm, feats_hbm, xs_v, ys_v, fcm_v,
                    *sets):
    (i00_0, i01_0, i10_0, i11_0, wx_0, wy_0, f00_0, f01_0, f10_0, f11_0,
     sem_0,
     i00_1, i01_1, i10_1, i11_1, wx_1, wy_1, f00_1, f01_1, f10_1, f11_1,
     sem_1) = sets
    buf = [
        (i00_0, i01_0, i10_0, i11_0, wx_0, wy_0, f00_0, f01_0, f10_0, f11_0,
         sem_0),
        (i00_1, i01_1, i10_1, i11_1, wx_1, wy_1, f00_1, f01_1, f10_1, f11_1,
         sem_1),
    ]
    wid = lax.axis_index("s") * _NC + lax.axis_index("c")
    b = wid // 2
    n_half = (wid % 2) * _HALF
    lanes = lax.iota(jnp.int32, _LANES)
    zeros16 = jnp.zeros((_LANES,), jnp.int32)

    def start(c, s):
        i00_v, i01_v, i10_v, i11_v, wx_v, wy_v, f00_v, f01_v, f10_v, f11_v, \
            sem = buf[s]
        n0 = n_half + c * _CH
        pltpu.sync_copy(coords_hbm.at[b, 0, pl.ds(n0, _CH)], xs_v)
        pltpu.sync_copy(coords_hbm.at[b, 1, pl.ds(n0, _CH)], ys_v)
        for g in range(_GROUPS):
            sl = pl.ds(g * _LANES, _LANES)
            xs = xs_v[sl] * jnp.float32(_W - 1)
            ys = ys_v[sl] * jnp.float32(_H - 1)
            x0 = jnp.clip(xs.astype(jnp.int32), 0, _W - 1)
            y0 = jnp.clip(ys.astype(jnp.int32), 0, _H - 1)
            wx_v[sl] = xs - x0.astype(jnp.float32)
            wy_v[sl] = ys - y0.astype(jnp.float32)
            x1 = jnp.minimum(x0 + 1, _W - 1)
            r0 = y0 * _W
            r1 = jnp.minimum(y0 + 1, _H - 1) * _W
            i00_v[sl] = r0 + x0
            i01_v[sl] = r0 + x1
            i10_v[sl] = r1 + x0
            i11_v[sl] = r1 + x1
        pltpu.async_copy(table_hbm.at[i00_v], f00_v, sem)
        pltpu.async_copy(table_hbm.at[i01_v], f01_v, sem)
        pltpu.async_copy(table_hbm.at[i10_v], f10_v, sem)
        pltpu.async_copy(table_hbm.at[i11_v], f11_v, sem)

    def finish(c, s):
        i00_v, i01_v, i10_v, i11_v, wx_v, wy_v, f00_v, f01_v, f10_v, f11_v, \
            sem = buf[s]
        pltpu.make_async_copy(table_hbm.at[i00_v], f00_v, sem).wait()
        pltpu.make_async_copy(table_hbm.at[i01_v], f01_v, sem).wait()
        pltpu.make_async_copy(table_hbm.at[i10_v], f10_v, sem).wait()
        pltpu.make_async_copy(table_hbm.at[i11_v], f11_v, sem).wait()

        def gbody(g, carry):
            sl = pl.ds(g * _LANES, _LANES)
            wx = wx_v[sl]
            wy = wy_v[sl]
            for j4 in range(_LANES // 16):
                rs = []
                for j in range(16 * j4, 16 * j4 + 16):
                    i = g * _LANES + j
                    jj = zeros16 + j
                    wxs = wx.at[jj].get(mode="promise_in_bounds")
                    wys = wy.at[jj].get(mode="promise_in_bounds")
                    iv = zeros16 + i
                    for cb in range(_F // _LANES):
                        csl = pl.ds(cb * _LANES, _LANES)
                        a = f00_v[i, csl]
                        bb = f01_v[i, csl]
                        d = f10_v[i, csl]
                        e = f11_v[i, csl]
                        top = a + wxs * (bb - a)
                        bot = d + wxs * (e - d)
                        rs.append((cb, iv, top + wys * (bot - top)))
                for cb, iv, r in rs:
                    plsc.store_scatter(fcm_v, [cb * _LANES + lanes, iv], r)
            return carry

        lax.fori_loop(0, _GROUPS, gbody, 0)
        tc = n_half // _CH + c
        pltpu.sync_copy(fcm_v.at[:, pl.ds(0, _CH)], feats_hbm.at[b, tc])

    start(0, 0)

    def body2(i, carry):
        c = 2 * i
        start(c + 1, 1)
        finish(c, 0)

        @pl.when(c + 2 < _NCHUNK)
        def _():
            start(c + 2, 0)

        finish(c + 1, 1)
        return carry

    lax.fori_loop(0, _NCHUNK // 2, body2, 0)


def _sc_sample(coords_t, table):
    set_types = []
    for _ in range(2):
        set_types += [
            pltpu.VMEM((_CH,), jnp.int32),
            pltpu.VMEM((_CH,), jnp.int32),
            pltpu.VMEM((_CH,), jnp.int32),
            pltpu.VMEM((_CH,), jnp.int32),
            pltpu.VMEM((_CH,), jnp.float32),
            pltpu.VMEM((_CH,), jnp.float32),
            pltpu.VMEM((_CH, _F), jnp.float32),
            pltpu.VMEM((_CH, _F), jnp.float32),
            pltpu.VMEM((_CH, _F), jnp.float32),
            pltpu.VMEM((_CH, _F), jnp.float32),
            pltpu.SemaphoreType.DMA,
        ]
    return pl.kernel(
        _sc_sample_body,
        out_type=jax.ShapeDtypeStruct((_NB, _N // _CH, _F, _CH),
                                      jnp.float32),
        mesh=plsc.VectorSubcoreMesh(core_axis_name="c", subcore_axis_name="s"),
        compiler_params=pltpu.CompilerParams(
            needs_layout_passes=False, use_tc_tiling_on_sc=False),
        scratch_types=[
            pltpu.VMEM((_CH,), jnp.float32),
            pltpu.VMEM((_CH,), jnp.float32),
            pltpu.VMEM((_F, _CH + 1), jnp.float32),
        ] + set_types,
    )(coords_t, table)


def _mlp_body(f_ref, w1t_ref, b1_ref, w2t_ref, b2_ref, o_ref):
    w1t = w1t_ref[...]
    w2t = w2t_ref[...]
    x2 = jnp.concatenate(
        [f_ref[0, t] for t in range(_TCB)], axis=1).astype(jnp.bfloat16)
    h = jnp.dot(w1t, x2, preferred_element_type=jnp.float32) + b1_ref[...]
    h = jnp.maximum(h, 0.0).astype(jnp.bfloat16)
    o_ref[0] = jnp.dot(w2t, h, preferred_element_type=jnp.float32) + b2_ref[...]


def _mlp(feats5, W1t, b1c, W2t, b2c):
    bn = _TCB * _CH
    return pl.pallas_call(
        _mlp_body,
        grid=(_NB, _N // bn),
        in_specs=[
            pl.BlockSpec((1, _TCB, _F, _CH), lambda i, j: (i, j, 0, 0)),
            pl.BlockSpec((_HID, _F), lambda i, j: (0, 0)),
            pl.BlockSpec((_HID, 1), lambda i, j: (0, 0)),
            pl.BlockSpec((_OUT, _HID), lambda i, j: (0, 0)),
            pl.BlockSpec((_OUT, 1), lambda i, j: (0, 0)),
        ],
        out_specs=pl.BlockSpec((1, _OUT, bn), lambda i, j: (i, 0, j)),
        out_shape=jax.ShapeDtypeStruct((_NB, _OUT, _N), jnp.float32),
    )(feats5, W1t, b1c, W2t, b2c)


def kernel(coords, feature_field, W1, b1, W2, b2):
    coords_t = jnp.transpose(coords, (0, 2, 1))
    table = feature_field.reshape(_H * _W, _F)
    feats5 = _sc_sample(coords_t, table)
    out3 = _mlp(feats5, W1.T.astype(jnp.bfloat16), b1.reshape(_HID, 1),
                W2.T.astype(jnp.bfloat16), b2.reshape(_OUT, 1))
    return jnp.transpose(out3, (0, 2, 1))


# speedup vs baseline: 1.0826x; 1.0826x over previous
import functools

import jax
import jax.numpy as jnp
from jax import lax
from jax.experimental import pallas as pl
from jax.experimental.pallas import tpu as pltpu
from jax.experimental.pallas import tpu_sc as plsc

_H = 1024
_W = 1024
_F = 64
_HID = 128
_OUT = 64
_NB = 16
_N = 65536

_NC = 2
_NS = 16
_NW = _NC * _NS
_LANES = 16

_CH = 128
_GROUPS = _CH // _LANES
_PER_TILE = _NB * _N // _NW
_NCHUNK = _PER_TILE // _CH
_HALF = _PER_TILE

_TCB = 32


def _sc_sample_body(coords_hbm, table_hbm, feats_hbm, xs_v, ys_v, fcm_v,
                    *sets):
    (i00_0, i01_0, i10_0, i11_0, wx_0, wy_0, f00_0, f01_0, f10_0, f11_0,
     sem_0,
     i00_1, i01_1, i10_1, i11_1, wx_1, wy_1, f00_1, f01_1, f10_1, f11_1,
     sem_1) = sets
    buf = [
        (i00_0, i01_0, i10_0, i11_0, wx_0, wy_0, f00_0, f01_0, f10_0, f11_0,
         sem_0),
        (i00_1, i01_1, i10_1, i11_1, wx_1, wy_1, f00_1, f01_1, f10_1, f11_1,
         sem_1),
    ]
    wid = lax.axis_index("s") * _NC + lax.axis_index("c")
    b = wid // 2
    n_half = (wid % 2) * _HALF
    lanes = lax.iota(jnp.int32, _LANES)
    zeros16 = jnp.zeros((_LANES,), jnp.int32)

    def start(c, s):
        i00_v, i01_v, i10_v, i11_v, wx_v, wy_v, f00_v, f01_v, f10_v, f11_v, \
            sem = buf[s]
        n0 = n_half + c * _CH
        pltpu.sync_copy(coords_hbm.at[b, 0, pl.ds(n0, _CH)], xs_v)
        pltpu.sync_copy(coords_hbm.at[b, 1, pl.ds(n0, _CH)], ys_v)
        for g in range(_GROUPS):
            sl = pl.ds(g * _LANES, _LANES)
            xs = xs_v[sl] * jnp.float32(_W - 1)
            ys = ys_v[sl] * jnp.float32(_H - 1)
            x0 = jnp.clip(xs.astype(jnp.int32), 0, _W - 1)
            y0 = jnp.clip(ys.astype(jnp.int32), 0, _H - 1)
            wx_v[sl] = xs - x0.astype(jnp.float32)
            wy_v[sl] = ys - y0.astype(jnp.float32)
            x1 = jnp.minimum(x0 + 1, _W - 1)
            r0 = y0 * _W
            r1 = jnp.minimum(y0 + 1, _H - 1) * _W
            i00_v[sl] = r0 + x0
            i01_v[sl] = r0 + x1
            i10_v[sl] = r1 + x0
            i11_v[sl] = r1 + x1
        pltpu.async_copy(table_hbm.at[i00_v], f00_v, sem)
        pltpu.async_copy(table_hbm.at[i01_v], f01_v, sem)
        pltpu.async_copy(table_hbm.at[i10_v], f10_v, sem)
        pltpu.async_copy(table_hbm.at[i11_v], f11_v, sem)

    def finish(c, s):
        i00_v, i01_v, i10_v, i11_v, wx_v, wy_v, f00_v, f01_v, f10_v, f11_v, \
            sem = buf[s]
        pltpu.make_async_copy(table_hbm.at[i00_v], f00_v, sem).wait()
        pltpu.make_async_copy(table_hbm.at[i01_v], f01_v, sem).wait()
        pltpu.make_async_copy(table_hbm.at[i10_v], f10_v, sem).wait()
        pltpu.make_async_copy(table_hbm.at[i11_v], f11_v, sem).wait()

        def gbody(g, carry):
            sl = pl.ds(g * _LANES, _LANES)
            wx = wx_v[sl]
            wy = wy_v[sl]
            for j4 in range(_LANES // 8):
                rs = []
                for j in range(8 * j4, 8 * j4 + 8):
                    i = g * _LANES + j
                    jj = zeros16 + j
                    wxs = wx.at[jj].get(mode="promise_in_bounds")
                    wys = wy.at[jj].get(mode="promise_in_bounds")
                    iv = zeros16 + i
                    for cb in range(_F // _LANES):
                        csl = pl.ds(cb * _LANES, _LANES)
                        a = f00_v[i, csl]
                        bb = f01_v[i, csl]
                        d = f10_v[i, csl]
                        e = f11_v[i, csl]
                        top = a + wxs * (bb - a)
                        bot = d + wxs * (e - d)
                        rs.append((cb, iv, top + wys * (bot - top)))
                for cb, iv, r in rs:
                    plsc.store_scatter(fcm_v, [cb * _LANES + lanes, iv], r)
            return carry

        lax.fori_loop(0, _GROUPS, gbody, 0)
        tc = n_half // _CH + c
        pltpu.sync_copy(fcm_v.at[:, pl.ds(0, _CH)], feats_hbm.at[b, tc])

    start(0, 0)

    def body2(i, carry):
        c = 2 * i
        start(c + 1, 1)
        finish(c, 0)

        @pl.when(c + 2 < _NCHUNK)
        def _():
            start(c + 2, 0)

        finish(c + 1, 1)
        return carry

    lax.fori_loop(0, _NCHUNK // 2, body2, 0)


def _sc_sample(coords_t, table):
    set_types = []
    for _ in range(2):
        set_types += [
            pltpu.VMEM((_CH,), jnp.int32),
            pltpu.VMEM((_CH,), jnp.int32),
            pltpu.VMEM((_CH,), jnp.int32),
            pltpu.VMEM((_CH,), jnp.int32),
            pltpu.VMEM((_CH,), jnp.float32),
            pltpu.VMEM((_CH,), jnp.float32),
            pltpu.VMEM((_CH, _F), jnp.float32),
            pltpu.VMEM((_CH, _F), jnp.float32),
            pltpu.VMEM((_CH, _F), jnp.float32),
            pltpu.VMEM((_CH, _F), jnp.float32),
            pltpu.SemaphoreType.DMA,
        ]
    return pl.kernel(
        _sc_sample_body,
        out_type=jax.ShapeDtypeStruct((_NB, _N // _CH, _F, _CH),
                                      jnp.float32),
        mesh=plsc.VectorSubcoreMesh(core_axis_name="c", subcore_axis_name="s"),
        compiler_params=pltpu.CompilerParams(
            needs_layout_passes=False, use_tc_tiling_on_sc=False),
        scratch_types=[
            pltpu.VMEM((_CH,), jnp.float32),
            pltpu.VMEM((_CH,), jnp.float32),
            pltpu.VMEM((_F, _CH + 1), jnp.float32),
        ] + set_types,
    )(coords_t, table)


def _mlp_body(f_ref, w1t_ref, b1_ref, w2t_ref, b2_ref, o_ref):
    w1t = w1t_ref[...]
    w2t = w2t_ref[...]
    x2 = jnp.concatenate(
        [f_ref[0, t] for t in range(_TCB)], axis=1).astype(jnp.bfloat16)
    h = jnp.dot(w1t, x2, preferred_element_type=jnp.float32) + b1_ref[...]
    h = jnp.maximum(h, 0.0).astype(jnp.bfloat16)
    o_ref[0] = jnp.dot(w2t, h, preferred_element_type=jnp.float32) + b2_ref[...]


def _mlp(feats5, W1t, b1c, W2t, b2c):
    bn = _TCB * _CH
    return pl.pallas_call(
        _mlp_body,
        grid=(_NB, _N // bn),
        in_specs=[
            pl.BlockSpec((1, _TCB, _F, _CH), lambda i, j: (i, j, 0, 0)),
            pl.BlockSpec((_HID, _F), lambda i, j: (0, 0)),
            pl.BlockSpec((_HID, 1), lambda i, j: (0, 0)),
            pl.BlockSpec((_OUT, _HID), lambda i, j: (0, 0)),
            pl.BlockSpec((_OUT, 1), lambda i, j: (0, 0)),
        ],
        out_specs=pl.BlockSpec((1, _OUT, bn), lambda i, j: (i, 0, j)),
        out_shape=jax.ShapeDtypeStruct((_NB, _OUT, _N), jnp.float32),
    )(feats5, W1t, b1c, W2t, b2c)


def kernel(coords, feature_field, W1, b1, W2, b2):
    coords_t = jnp.transpose(coords, (0, 2, 1))
    table = feature_field.reshape(_H * _W, _F)
    feats5 = _sc_sample(coords_t, table)
    out3 = _mlp(feats5, W1.T.astype(jnp.bfloat16), b1.reshape(_HID, 1),
                W2.T.astype(jnp.bfloat16), b2.reshape(_OUT, 1))
    return jnp.transpose(out3, (0, 2, 1))
